# baseline (speedup 1.0000x reference)
import functools

import jax
import jax.numpy as jnp
from jax import lax
from jax.experimental import pallas as pl
from jax.experimental.pallas import tpu as pltpu
from jax.experimental.pallas import tpu_sc as plsc

ROWS = 128
N = 32768
K = 8
LANES = 16


SC_ROWS = 48
NB = 16
VPB = N // (NB * LANES)
BLK = VPB * LANES
NW = 16
RPW = SC_ROWS // NW


def _merge_kv(a, ia, b, ib):
    rb = lax.rev(b, (0,))
    rib = lax.rev(ib, (0,))
    keep_a = a >= rb
    m = jnp.maximum(a, rb)
    im = jnp.where(keep_a, ia, rib)
    return plsc.sort_key_val(m, im)


def _merge_v(a, b):
    m = jnp.maximum(a, lax.rev(b, (0,)))
    return lax.sort(m, dimension=0)


def _sc_body(x_hbm, o_hbm, row_a, row_b, ob_v, sem_a, sem_b):
    wid = lax.axis_index("s")
    iota = lax.iota(jnp.int32, LANES)
    neg = jnp.full((LANES,), -jnp.inf, dtype=jnp.float32)

    bufs = (row_a, row_b)
    sems = (sem_a, sem_b)
    row0 = wid * RPW
    pltpu.async_copy(x_hbm.at[row0], row_a, sem_a)

    for r in range(RPW):
        row_v = bufs[r % 2]
        pltpu.make_async_copy(x_hbm.at[row0 + r], row_v, sems[r % 2]).wait()
        if r + 1 < RPW:
            pltpu.async_copy(
                x_hbm.at[row0 + r + 1], bufs[(r + 1) % 2], sems[(r + 1) % 2])

        def s1(j, accs):
            off = j * LANES
            return tuple(
                jnp.maximum(accs[b], row_v[pl.ds(b * BLK + off, LANES)])
                for b in range(NB))

        accs = lax.fori_loop(0, VPB, s1, (neg,) * NB, unroll=2)

        kvs = [plsc.sort_key_val(accs[b], iota + b * BLK) for b in range(NB)]
        while len(kvs) > 1:
            kvs = [_merge_kv(kvs[i][0], kvs[i][1], kvs[i + 1][0], kvs[i + 1][1])
                   for i in range(0, len(kvs), 2)]
        ids = kvs[0][1]

        top = [neg] * K
        for i in range(K):
            base = ids[LANES - 1 - i]
            for g in range(VPB // LANES):
                idx = base + g * (LANES * LANES) + iota * LANES
                v = plsc.load_gather(row_v, [idx])
                for t in range(K):
                    hi = jnp.maximum(top[t], v)
                    v = jnp.minimum(top[t], v)
                    top[t] = hi

        vs = [lax.sort(t, dimension=0) for t in top]
        while len(vs) > 1:
            vs = [_merge_v(vs[i], vs[i + 1]) for i in range(0, len(vs), 2)]
        ob_v[...] = lax.rev(vs[0], (0,))
        pltpu.sync_copy(ob_v, o_hbm.at[row0 + r])


RPB = 8
NV = N // 128
GRP = 8

SORT8 = [(0, 1), (2, 3), (4, 5), (6, 7),
         (0, 2), (1, 3), (4, 6), (5, 7),
         (1, 2), (5, 6), (0, 4), (3, 7),
         (1, 5), (2, 6), (3, 5), (2, 4),
         (1, 2), (3, 4), (5, 6)]


def _sort8_desc(vs):
    vs = list(vs)
    for a, b in SORT8:
        hi = jnp.maximum(vs[a], vs[b])
        lo = jnp.minimum(vs[a], vs[b])
        vs[a], vs[b] = hi, lo
    return vs


def _merge8_desc(t, s):
    l = [jnp.maximum(t[i], s[7 - i]) for i in range(8)]
    for d in (4, 2, 1):
        nxt = list(l)
        for i in range(8):
            if (i // d) % 2 == 0:
                nxt[i] = jnp.maximum(l[i], l[i + d])
            else:
                nxt[i] = jnp.minimum(l[i], l[i - d])
        l = nxt
    return l


def _tc_body(x_ref, o_ref):
    runs = []
    for c0 in range(0, NV, GRP):
        vs = [x_ref[:, pl.ds((c0 + j) * 128, 128)] for j in range(GRP)]
        runs.append(_sort8_desc(vs))
    while len(runs) > 1:
        runs = [_merge8_desc(runs[i], runs[i + 1])
                for i in range(0, len(runs), 2)]
    t = runs[0]

    lane = lax.broadcasted_iota(jnp.int32, (RPB, 128), 1)
    big = jnp.int32(1 << 30)
    outs = []
    for _ in range(K):
        m = jnp.max(t[0], axis=1, keepdims=True)
        am = jnp.min(jnp.where(t[0] == m, lane, big), axis=1, keepdims=True)
        onehot = lane == am
        for i in range(K - 1):
            t[i] = jnp.where(onehot, t[i + 1], t[i])
        t[K - 1] = jnp.where(onehot, -jnp.inf, t[K - 1])
        outs.append(m)
    o_ref[...] = jnp.concatenate(outs, axis=1)


@jax.jit
def kernel(x):
    sc_f = pl.kernel(
        _sc_body,
        out_type=jax.ShapeDtypeStruct((SC_ROWS, LANES), jnp.float32),
        mesh=plsc.VectorSubcoreMesh(
            core_axis_name="c", subcore_axis_name="s", num_cores=1),
        compiler_params=pltpu.CompilerParams(needs_layout_passes=False),
        scratch_types=[
            pltpu.VMEM((N,), jnp.float32),
            pltpu.VMEM((N,), jnp.float32),
            pltpu.VMEM((LANES,), jnp.float32),
            pltpu.SemaphoreType.DMA,
            pltpu.SemaphoreType.DMA,
        ],
    )
    sc_out = sc_f(x[:SC_ROWS])[:, :K]

    tc_rows = ROWS - SC_ROWS
    tc_out = pl.pallas_call(
        _tc_body,
        grid=(tc_rows // RPB,),
        in_specs=[pl.BlockSpec((RPB, N), lambda i: (i, 0))],
        out_specs=pl.BlockSpec((RPB, K), lambda i: (i, 0)),
        out_shape=jax.ShapeDtypeStruct((tc_rows, K), jnp.float32),
    )(x[SC_ROWS:])

    return jnp.concatenate([sc_out, tc_out], axis=0)


# speedup vs baseline: 1.5231x; 1.5231x over previous
import functools

import jax
import jax.numpy as jnp
from jax import lax
from jax.experimental import pallas as pl
from jax.experimental.pallas import tpu as pltpu
from jax.experimental.pallas import tpu_sc as plsc

ROWS = 128
N = 32768
K = 8
LANES = 16
NB = 16
VPB = N // (NB * LANES)
BLK = VPB * LANES
NW = 32
RPW = ROWS // NW


def _merge_kv(a, ia, b, ib):
    rb = lax.rev(b, (0,))
    rib = lax.rev(ib, (0,))
    keep_a = a >= rb
    m = jnp.maximum(a, rb)
    im = jnp.where(keep_a, ia, rib)
    return plsc.sort_key_val(m, im)


def _merge_v(a, b):
    m = jnp.maximum(a, lax.rev(b, (0,)))
    return lax.sort(m, dimension=0)


def _sc_body(x_hbm, o_hbm, row_a, row_b, ob_v, sem_a, sem_b):
    wid = lax.axis_index("s") * 2 + lax.axis_index("c")
    iota = lax.iota(jnp.int32, LANES)
    neg = jnp.full((LANES,), -jnp.inf, dtype=jnp.float32)

    bufs = (row_a, row_b)
    sems = (sem_a, sem_b)
    row0 = wid * RPW
    pltpu.async_copy(x_hbm.at[row0], row_a, sem_a)

    for r in range(RPW):
        row_v = bufs[r % 2]
        pltpu.make_async_copy(x_hbm.at[row0 + r], row_v, sems[r % 2]).wait()
        if r + 1 < RPW:
            pltpu.async_copy(
                x_hbm.at[row0 + r + 1], bufs[(r + 1) % 2], sems[(r + 1) % 2])

        def s1(j, accs):
            off = j * LANES
            return tuple(
                jnp.maximum(accs[b], row_v[pl.ds(b * BLK + off, LANES)])
                for b in range(NB))

        accs = lax.fori_loop(0, VPB, s1, (neg,) * NB, unroll=2)

        kvs = [plsc.sort_key_val(accs[b], iota + b * BLK) for b in range(NB)]
        while len(kvs) > 1:
            kvs = [_merge_kv(kvs[i][0], kvs[i][1], kvs[i + 1][0], kvs[i + 1][1])
                   for i in range(0, len(kvs), 2)]
        ids = kvs[0][1]

        top = [neg] * K
        for i in range(K):
            base = ids[LANES - 1 - i]
            for g in range(VPB // LANES):
                idx = base + g * (LANES * LANES) + iota * LANES
                v = plsc.load_gather(row_v, [idx])
                for t in range(K):
                    hi = jnp.maximum(top[t], v)
                    v = jnp.minimum(top[t], v)
                    top[t] = hi

        vs = [lax.sort(t, dimension=0) for t in top]
        while len(vs) > 1:
            vs = [_merge_v(vs[i], vs[i + 1]) for i in range(0, len(vs), 2)]
        ob_v[...] = lax.rev(vs[0], (0,))
        pltpu.sync_copy(ob_v.at[pl.ds(0, K)],
                        o_hbm.at[pl.ds((row0 + r) * K, K)])


@jax.jit
def kernel(x):
    f = pl.kernel(
        _sc_body,
        out_type=jax.ShapeDtypeStruct((ROWS * K,), jnp.float32),
        mesh=plsc.VectorSubcoreMesh(core_axis_name="c", subcore_axis_name="s"),
        compiler_params=pltpu.CompilerParams(needs_layout_passes=False),
        scratch_types=[
            pltpu.VMEM((N,), jnp.float32),
            pltpu.VMEM((N,), jnp.float32),
            pltpu.VMEM((LANES,), jnp.float32),
            pltpu.SemaphoreType.DMA,
            pltpu.SemaphoreType.DMA,
        ],
    )
    return f(x).reshape(ROWS, K)
